# baseline (speedup 1.0000x reference)
import jax
import jax.numpy as jnp
from jax import lax
from jax.experimental import pallas as pl

N_SNPS = 20000
D_DELTA = 768
D_GENE = 200
D_MODEL = 64
BATCH = 32
N_GENO = 4
EPS = 1e-5

_S = 4096
_BG = 8


def _body(x_blk, delta, gene, wseq, wfunc, bseq, bfunc, table, gamma, beta, out):
    bio = lax.dot_general(wseq[...], delta[...], (((1,), (1,)), ((), ())),
                          preferred_element_type=jnp.float32)
    bio = bio + lax.dot_general(wfunc[...], gene[...], (((1,), (1,)), ((), ())),
                                preferred_element_type=jnp.float32)
    bio = bio + (bseq[...] + bfunc[...]).reshape(D_MODEL, 1)
    g_col = gamma[...].reshape(D_MODEL, 1)
    b_col = beta[...].reshape(D_MODEL, 1)
    a = []
    for g in range(N_GENO):
        t = bio + table[...][g, :].reshape(D_MODEL, 1)
        mu = jnp.mean(t, axis=0, keepdims=True)
        var = jnp.mean(jnp.square(t - mu), axis=0, keepdims=True)
        a.append((t - mu) * lax.rsqrt(var + EPS) * g_col + b_col)
    x = x_blk[...]
    for b in range(_BG):
        xb = x[b, :].reshape(1, _S)
        sel = jnp.where(xb == 2, a[2], a[3])
        sel = jnp.where(xb == 1, a[1], sel)
        sel = jnp.where(xb == 0, a[0], sel)
        out[b] = sel


def kernel(x_cat, delta_E, gene_E, W_seq, b_seq, W_func, b_func, geno_table, gamma, beta):
    grid = (pl.cdiv(N_SNPS, _S), BATCH // _BG)
    full = lambda shape: pl.BlockSpec(shape, lambda i, j: tuple(0 for _ in shape))
    out_t = pl.pallas_call(
        _body,
        grid=grid,
        in_specs=[
            pl.BlockSpec((_BG, _S), lambda i, j: (j, i)),
            pl.BlockSpec((_S, D_DELTA), lambda i, j: (i, 0)),
            pl.BlockSpec((_S, D_GENE), lambda i, j: (i, 0)),
            full((D_MODEL, D_DELTA)),
            full((D_MODEL, D_GENE)),
            full((1, D_MODEL)),
            full((1, D_MODEL)),
            full((N_GENO, D_MODEL)),
            full((1, D_MODEL)),
            full((1, D_MODEL)),
        ],
        out_specs=pl.BlockSpec((_BG, D_MODEL, _S), lambda i, j: (j, 0, i)),
        out_shape=jax.ShapeDtypeStruct((BATCH, D_MODEL, N_SNPS), jnp.float32),
    )(x_cat, delta_E, gene_E, W_seq, W_func,
      b_seq.reshape(1, D_MODEL), b_func.reshape(1, D_MODEL), geno_table,
      gamma.reshape(1, D_MODEL), beta.reshape(1, D_MODEL))
    return jnp.transpose(out_t, (0, 2, 1))


# speedup vs baseline: 1.6333x; 1.6333x over previous
import jax
import jax.numpy as jnp
from jax import lax
from jax.experimental import pallas as pl

N_SNPS = 20000
D_DELTA = 768
D_GENE = 200
D_MODEL = 64
BATCH = 32
N_GENO = 4
EPS = 1e-5

_S = 2048


def _body(x_blk, delta, gene, wseq, wfunc, bseq, bfunc, table, gamma, beta, out):
    bio = lax.dot_general(wseq[...], delta[...], (((1,), (1,)), ((), ())),
                          preferred_element_type=jnp.float32)
    bio = bio + lax.dot_general(wfunc[...], gene[...], (((1,), (1,)), ((), ())),
                                preferred_element_type=jnp.float32)
    bio = bio + (bseq[...] + bfunc[...]).reshape(D_MODEL, 1)
    g_col = gamma[...].reshape(D_MODEL, 1)
    b_col = beta[...].reshape(D_MODEL, 1)
    a = []
    for g in range(N_GENO):
        t = bio + table[...][g, :].reshape(D_MODEL, 1)
        mu = jnp.mean(t, axis=0, keepdims=True)
        var = jnp.mean(jnp.square(t - mu), axis=0, keepdims=True)
        a.append((t - mu) * lax.rsqrt(var + EPS) * g_col + b_col)
    x = x_blk[...]
    for b in range(BATCH):
        xb = x[b, :].reshape(1, _S)
        sel = jnp.where(xb == 2, a[2], a[3])
        sel = jnp.where(xb == 1, a[1], sel)
        sel = jnp.where(xb == 0, a[0], sel)
        out[b] = sel


def kernel(x_cat, delta_E, gene_E, W_seq, b_seq, W_func, b_func, geno_table, gamma, beta):
    grid = (pl.cdiv(N_SNPS, _S),)
    full = lambda shape: pl.BlockSpec(shape, lambda i: tuple(0 for _ in shape))
    out_t = pl.pallas_call(
        _body,
        grid=grid,
        in_specs=[
            pl.BlockSpec((BATCH, _S), lambda i: (0, i)),
            pl.BlockSpec((_S, D_DELTA), lambda i: (i, 0)),
            pl.BlockSpec((_S, D_GENE), lambda i: (i, 0)),
            full((D_MODEL, D_DELTA)),
            full((D_MODEL, D_GENE)),
            full((1, D_MODEL)),
            full((1, D_MODEL)),
            full((N_GENO, D_MODEL)),
            full((1, D_MODEL)),
            full((1, D_MODEL)),
        ],
        out_specs=pl.BlockSpec((BATCH, D_MODEL, _S), lambda i: (0, 0, i)),
        out_shape=jax.ShapeDtypeStruct((BATCH, D_MODEL, N_SNPS), jnp.float32),
    )(x_cat, delta_E, gene_E, W_seq, W_func,
      b_seq.reshape(1, D_MODEL), b_func.reshape(1, D_MODEL), geno_table,
      gamma.reshape(1, D_MODEL), beta.reshape(1, D_MODEL))
    return jnp.transpose(out_t, (0, 2, 1))
